# baseline (speedup 1.0000x reference)
import functools

import jax
import jax.numpy as jnp
from jax import lax
from jax.experimental import pallas as pl
from jax.experimental.pallas import tpu as pltpu
from jax.experimental.pallas import tpu_sc as plsc

NN = 10000
DD = 256
DH = 128
QD = 64
NQ = DD // QD
HH = 1024
EE = 160000
NC = 2
NS = 16
NP = 10240
RPT = NP // NS
KC = 128
EP = 163840
CPT = EP // (NS * KC)
NBUF = 4
PD = 2


@functools.cache
def _mesh():
  return plsc.VectorSubcoreMesh(
      core_axis_name="c", subcore_axis_name="s", num_cores=NC, num_subcores=NS)


def _agg_body(do_deg, x4, src_g2, dst_g2, src_t2, dst_t2, w_t2,
              gsum4, tsum4, deg_out,
              acc_sh, idx_all, dst_all, w_all,
              r0, r1, r2, r3, sg0, sg1, sg2, sg3, ss0, ss1, ss2, ss3):
  rows = (r0, r1, r2, r3)
  semg = (sg0, sg1, sg2, sg3)
  sems = (ss0, ss1, ss2, ss3)
  cid = lax.axis_index("c")
  sid = lax.axis_index("s")
  rbase = sid * RPT
  cbase = sid * CPT

  def fill(buf, val):
    v = jnp.full((16,), val, jnp.float32)

    def row(e, _):
      for j in range(QD // 16):
        buf[e, pl.ds(j * 16, 16)] = v
      return 0

    lax.fori_loop(0, KC, row, 0)

  def zero_acc():
    fill(rows[0], 0.0)
    for i in range(RPT // KC):
      pltpu.sync_copy(rows[0], acc_sh.at[pl.ds(rbase + i * KC, KC)])

  def flush(out, q):
    pltpu.sync_copy(
        acc_sh.at[pl.ds(rbase, RPT)],
        out.at[cid].at[pl.ds(rbase, RPT), pl.ds(q * QD, QD)])

  def scale_buf(buf, c):
    def group(g, _):
      w16 = w_all[c, pl.ds(g * 16, 16)]
      for lane in range(16):
        w = w16[lane]
        for j in range(QD // 16):
          buf[g * 16 + lane, pl.ds(j * 16, 16)] = (
              buf[g * 16 + lane, pl.ds(j * 16, 16)] * w)
      return 0

    lax.fori_loop(0, KC // 16, group, 0)

  def wait_gather(xq, b):
    pltpu.make_async_copy(xq.at[idx_all.at[0]], rows[b], semg[b]).wait()

  def wait_scatter(b):
    pltpu.make_async_copy(rows[b], acc_sh.at[dst_all.at[0]], sems[b]).wait()

  def quarter_pass(xq, scale):
    for b in range(PD):
      pltpu.async_copy(xq.at[idx_all.at[b]], rows[b], semg[b])

    def quad(q, _):
      for b in range(NBUF):
        c = q * NBUF + b
        wait_gather(xq, b)
        if scale:
          scale_buf(rows[b], c)
        pltpu.async_copy(rows[b], acc_sh.at[dst_all.at[c]], sems[b],
                         add=True)
        cp = c + PD
        bp = (b + PD) % NBUF

        @pl.when(cp < CPT)
        def _():
          @pl.when(cp >= NBUF)
          def _():
            wait_scatter(bp)
          pltpu.async_copy(xq.at[idx_all.at[cp]], rows[bp], semg[bp])
      return 0

    lax.fori_loop(0, CPT // NBUF, quad, 0)
    for b in range(NBUF):
      wait_scatter(b)

  def relation(src2, dst2, out, scale):
    pltpu.sync_copy(src2.at[pl.ds(cbase, CPT)], idx_all)
    pltpu.sync_copy(dst2.at[pl.ds(cbase, CPT)], dst_all)
    if scale:
      pltpu.sync_copy(w_t2.at[pl.ds(cbase, CPT)], w_all)
    for q in range(NC):
      qidx = cid * NC + q
      zero_acc()
      plsc.subcore_barrier()
      quarter_pass(x4.at[qidx], scale)
      plsc.subcore_barrier()
      flush(out, q)
      plsc.subcore_barrier()

  relation(src_g2, dst_g2, gsum4, False)

  if do_deg:
    zero_acc()
    fill(rows[1], 1.0)
    plsc.subcore_barrier()

    def dchunk(i, _):
      c = cid * (CPT // 2) + i

      @pl.when(i >= 2)
      def _():
        wait_scatter(1)

      pltpu.async_copy(rows[1], acc_sh.at[dst_all.at[c]], sems[1], add=True)
      return 0

    lax.fori_loop(0, CPT // 2, dchunk, 0)
    wait_scatter(1)
    wait_scatter(1)
    plsc.subcore_barrier()
    flush(deg_out, 0)
    plsc.subcore_barrier()

  relation(src_t2, dst_t2, tsum4, True)


def _make_agg(do_deg):
  out_type = [
      jax.ShapeDtypeStruct((NC, NP, DH), jnp.float32),
      jax.ShapeDtypeStruct((NC, NP, DH), jnp.float32),
      jax.ShapeDtypeStruct((NC, NP, DH), jnp.float32),
  ]
  if not do_deg:
    out_type = out_type[:2]
  scratch = (
      [pltpu.VMEM_SHARED((NP, QD), jnp.float32)]
      + [pltpu.VMEM((CPT, KC), jnp.int32)] * 2
      + [pltpu.VMEM((CPT, KC), jnp.float32)]
      + [pltpu.VMEM((KC, QD), jnp.float32)] * NBUF
      + [pltpu.SemaphoreType.DMA] * (2 * NBUF)
  )

  if do_deg:
    def body(x4, src_g2, dst_g2, src_t2, dst_t2, w_t2, gsum4, tsum4,
             deg_out, *scr):
      _agg_body(True, x4, src_g2, dst_g2, src_t2, dst_t2, w_t2,
                gsum4, tsum4, deg_out, *scr)
  else:
    def body(x4, src_g2, dst_g2, src_t2, dst_t2, w_t2, gsum4, tsum4, *scr):
      _agg_body(False, x4, src_g2, dst_g2, src_t2, dst_t2, w_t2,
                gsum4, tsum4, None, *scr)

  return pl.kernel(body, out_type=out_type, mesh=_mesh(),
                   scratch_types=scratch, name="sc_agg",
                   compiler_params=pltpu.CompilerParams(
                       use_tc_tiling_on_sc=False))


_agg_deg = lambda *a: _make_agg_cached(True)(*a)
_agg = lambda *a: _make_agg_cached(False)(*a)
_make_agg_cached = functools.cache(_make_agg)

TT = 400
GRID = NN // TT


def _dense_body(x_lo, x_hi, g_lo, g_hi, t_lo, t_hi, deg_a, deg_b, w1, b1,
                geo_out, hsum_out):
  i = pl.program_id(0)
  x = jnp.concatenate([x_lo[0], x_hi[0]], axis=1)
  gs = jnp.concatenate([g_lo[0], g_hi[0]], axis=1)
  ts = jnp.concatenate([t_lo[0], t_hi[0]], axis=1)
  invd = 1.0 / (deg_a[0, :, 0:1] + deg_b[0, :, 0:1] + 1.0)
  geo = (gs + x) * invd
  geo_out[...] = geo
  hg = jnp.tanh(jnp.dot(geo, w1[...], preferred_element_type=jnp.float32)
                + b1[...])
  ht = jnp.tanh(jnp.dot(ts, w1[...], preferred_element_type=jnp.float32)
                + b1[...])
  s = jnp.concatenate([jnp.sum(hg, 0, keepdims=True),
                       jnp.sum(ht, 0, keepdims=True)], axis=0)

  @pl.when(i == 0)
  def _():
    hsum_out[...] = s

  @pl.when(i > 0)
  def _():
    hsum_out[...] += s


def _half(c):
  return pl.BlockSpec((1, TT, DH), lambda i, c=c: (c, i, 0))


def _dense(x2, gsum2, tsum2, deg2, w1, b1r):
  return pl.pallas_call(
      _dense_body,
      grid=(GRID,),
      in_specs=[_half(0), _half(1), _half(0), _half(1), _half(0), _half(1),
                _half(0), _half(1),
                pl.BlockSpec((DD, HH), lambda i: (0, 0)),
                pl.BlockSpec((1, HH), lambda i: (0, 0))],
      out_specs=[pl.BlockSpec((TT, DD), lambda i: (i, 0)),
                 pl.BlockSpec((2, HH), lambda i: (0, 0))],
      out_shape=[jax.ShapeDtypeStruct((NN, DD), jnp.float32),
                 jax.ShapeDtypeStruct((2, HH), jnp.float32)],
  )(x2, x2, gsum2, gsum2, tsum2, tsum2, deg2, deg2, w1, b1r)


def _combine_body(emit_q, hsum, w2r, geo, t_lo, t_hi, *outs):
  wm = jnp.sum(hsum[...] * w2r[...], axis=1) / NN
  m = jnp.max(wm)
  e = jnp.exp(wm - m)
  beta = e / jnp.sum(e)
  g = geo[...]
  lo = beta[0] * g[:, :DH] + beta[1] * t_lo[0]
  hi = beta[0] * g[:, DH:] + beta[1] * t_hi[0]
  outs[0][0] = lo
  outs[0][1] = hi
  if emit_q:
    outs[1][0] = lo[:, :QD]
    outs[1][1] = lo[:, QD:]
    outs[1][2] = hi[:, :QD]
    outs[1][3] = hi[:, QD:]


def _combine(hsum, w2r, geo, tsum2, emit_q):
  out_specs = [pl.BlockSpec((NC, TT, DH), lambda i: (0, i, 0))]
  out_shape = [jax.ShapeDtypeStruct((NC, NN, DH), jnp.float32)]
  if emit_q:
    out_specs.append(pl.BlockSpec((NQ, TT, QD), lambda i: (0, i, 0)))
    out_shape.append(jax.ShapeDtypeStruct((NQ, NN, QD), jnp.float32))
  return pl.pallas_call(
      functools.partial(_combine_body, emit_q),
      grid=(GRID,),
      in_specs=[pl.BlockSpec((2, HH), lambda i: (0, 0)),
                pl.BlockSpec((1, HH), lambda i: (0, 0)),
                pl.BlockSpec((TT, DD), lambda i: (i, 0)),
                _half(0), _half(1)],
      out_specs=out_specs,
      out_shape=out_shape,
  )(hsum, w2r, geo, tsum2, tsum2)


def kernel(loc_feat, geo_edge_index, trans_edge_index, trans_w,
           W1_0, b1_0, W2_0, W1_1, b1_1, W2_1):
  npad = EP - EE
  pad_src = jnp.arange(npad, dtype=jnp.int32) % NN
  pad_dst = NN + jnp.arange(npad, dtype=jnp.int32) % (NP - NN)

  def prep(ei):
    s = jnp.concatenate([ei[0], pad_src]).reshape(EP // KC, KC)
    d = jnp.concatenate([ei[1], pad_dst]).reshape(EP // KC, KC)
    return s, d

  src_g2, dst_g2 = prep(geo_edge_index)
  src_t2, dst_t2 = prep(trans_edge_index)
  w_t2 = jnp.concatenate(
      [trans_w, jnp.zeros((npad,), jnp.float32)]).reshape(EP // KC, KC)
  x4 = jnp.stack([loc_feat[:, q * QD:(q + 1) * QD] for q in range(NQ)])
  x2 = jnp.stack([loc_feat[:, :DH], loc_feat[:, DH:]])
  b1_0r = b1_0.reshape(1, HH)
  b1_1r = b1_1.reshape(1, HH)
  w2_0r = W2_0.reshape(1, HH)
  w2_1r = W2_1.reshape(1, HH)

  gsum2, tsum2, deg2 = _agg_deg(x4, src_g2, dst_g2, src_t2, dst_t2, w_t2)
  geo1, hsum1 = _dense(x2, gsum2, tsum2, deg2, W1_0, b1_0r)
  x2, x4 = _combine(hsum1, w2_0r, geo1, tsum2, True)

  gsum2, tsum2 = _agg(x4, src_g2, dst_g2, src_t2, dst_t2, w_t2)
  geo2, hsum2 = _dense(x2, gsum2, tsum2, deg2, W1_1, b1_1r)
  (x2,) = _combine(hsum2, w2_1r, geo2, tsum2, False)

  return jnp.moveaxis(x2, 0, 1).reshape(NN, DD)


# speedup vs baseline: 1.2202x; 1.2202x over previous
import functools

import jax
import jax.numpy as jnp
from jax import lax
from jax.experimental import pallas as pl
from jax.experimental.pallas import tpu as pltpu
from jax.experimental.pallas import tpu_sc as plsc

NN = 10000
DD = 256
DH = 128
QD = 64
NQ = DD // QD
HH = 1024
EE = 160000
NC = 2
NS = 16
NP = 10240
RPT = NP // NS
KC = 128
EP = 163840
CPT = EP // (NS * KC)
NBUF = 4
PD = 2


@functools.cache
def _mesh():
  return plsc.VectorSubcoreMesh(
      core_axis_name="c", subcore_axis_name="s", num_cores=NC, num_subcores=NS)


def _agg_body(do_deg, x4, src_g2, dst_g2, src_t2, dst_t2, w_t2,
              gsum4, tsum4, deg_out,
              acc_sh, idx_all, dst_all, w_all,
              r0, r1, r2, r3, sg0, sg1, sg2, sg3, ss0, ss1, ss2, ss3):
  rows = (r0, r1, r2, r3)
  semg = (sg0, sg1, sg2, sg3)
  sems = (ss0, ss1, ss2, ss3)
  cid = lax.axis_index("c")
  sid = lax.axis_index("s")
  rbase = sid * RPT
  cbase = sid * CPT

  def fill(buf, val):
    v = jnp.full((16,), val, jnp.float32)

    def row(e, _):
      for j in range(QD // 16):
        buf[e, pl.ds(j * 16, 16)] = v
      return 0

    lax.fori_loop(0, KC, row, 0)

  def zero_acc():
    fill(rows[0], 0.0)
    for i in range(RPT // KC):
      pltpu.sync_copy(rows[0], acc_sh.at[pl.ds(rbase + i * KC, KC)])

  def flush(out, q):
    pltpu.sync_copy(
        acc_sh.at[pl.ds(rbase, RPT)],
        out.at[cid].at[pl.ds(rbase, RPT), pl.ds(q * QD, QD)])

  def scale_buf(buf, c):
    def group(g, _):
      w16 = w_all[c, pl.ds(g * 16, 16)]
      for lane in range(16):
        w = w16[lane]
        for j in range(QD // 16):
          buf[g * 16 + lane, pl.ds(j * 16, 16)] = (
              buf[g * 16 + lane, pl.ds(j * 16, 16)] * w)
      return 0

    lax.fori_loop(0, KC // 16, group, 0)

  def wait_gather(xq, b):
    pltpu.make_async_copy(xq.at[idx_all.at[0]], rows[b], semg[b]).wait()

  def wait_scatter(b):
    pltpu.make_async_copy(rows[b], acc_sh.at[dst_all.at[0]], sems[b]).wait()

  def quarter_pass(xq, scale):
    for b in range(PD):
      pltpu.async_copy(xq.at[idx_all.at[b]], rows[b], semg[b])

    def quad(q, _):
      for b in range(NBUF):
        c = q * NBUF + b
        wait_gather(xq, b)
        if scale:
          scale_buf(rows[b], c)
        pltpu.async_copy(rows[b], acc_sh.at[dst_all.at[c]], sems[b],
                         add=True)
        cp = c + PD
        bp = (b + PD) % NBUF

        @pl.when(cp < CPT)
        def _():
          @pl.when(cp >= NBUF)
          def _():
            wait_scatter(bp)
          pltpu.async_copy(xq.at[idx_all.at[cp]], rows[bp], semg[bp])
      return 0

    lax.fori_loop(0, CPT // NBUF, quad, 0)
    for b in range(NBUF):
      wait_scatter(b)

  def relation(src2, dst2, out, scale):
    pltpu.sync_copy(src2.at[pl.ds(cbase, CPT)], idx_all)
    pltpu.sync_copy(dst2.at[pl.ds(cbase, CPT)], dst_all)
    if scale:
      pltpu.sync_copy(w_t2.at[pl.ds(cbase, CPT)], w_all)
    for q in range(NC):
      qidx = cid * NC + q
      zero_acc()
      plsc.subcore_barrier()
      quarter_pass(x4.at[qidx], scale)
      plsc.subcore_barrier()
      flush(out, q)
      plsc.subcore_barrier()

  relation(src_g2, dst_g2, gsum4, False)

  if do_deg:
    zero_acc()
    fill(rows[1], 1.0)
    plsc.subcore_barrier()

    def dchunk(i, _):
      c = cid * (CPT // 2) + i

      @pl.when(i >= 2)
      def _():
        wait_scatter(1)

      pltpu.async_copy(rows[1], acc_sh.at[dst_all.at[c]], sems[1], add=True)
      return 0

    lax.fori_loop(0, CPT // 2, dchunk, 0)
    wait_scatter(1)
    wait_scatter(1)
    plsc.subcore_barrier()
    flush(deg_out, 0)
    plsc.subcore_barrier()

  relation(src_t2, dst_t2, tsum4, True)


def _make_agg(do_deg):
  out_type = [
      jax.ShapeDtypeStruct((NC, NP, DH), jnp.float32),
      jax.ShapeDtypeStruct((NC, NP, DH), jnp.float32),
      jax.ShapeDtypeStruct((NC, NP, DH), jnp.float32),
  ]
  if not do_deg:
    out_type = out_type[:2]
  scratch = (
      [pltpu.VMEM_SHARED((NP, QD), jnp.float32)]
      + [pltpu.VMEM((CPT, KC), jnp.int32)] * 2
      + [pltpu.VMEM((CPT, KC), jnp.float32)]
      + [pltpu.VMEM((KC, QD), jnp.float32)] * NBUF
      + [pltpu.SemaphoreType.DMA] * (2 * NBUF)
  )

  if do_deg:
    def body(x4, src_g2, dst_g2, src_t2, dst_t2, w_t2, gsum4, tsum4,
             deg_out, *scr):
      _agg_body(True, x4, src_g2, dst_g2, src_t2, dst_t2, w_t2,
                gsum4, tsum4, deg_out, *scr)
  else:
    def body(x4, src_g2, dst_g2, src_t2, dst_t2, w_t2, gsum4, tsum4, *scr):
      _agg_body(False, x4, src_g2, dst_g2, src_t2, dst_t2, w_t2,
                gsum4, tsum4, None, *scr)

  return pl.kernel(body, out_type=out_type, mesh=_mesh(),
                   scratch_types=scratch, name="sc_agg",
                   compiler_params=pltpu.CompilerParams(
                       use_tc_tiling_on_sc=False))


_agg_deg = lambda *a: _make_agg_cached(True)(*a)
_agg = lambda *a: _make_agg_cached(False)(*a)
_make_agg_cached = functools.cache(_make_agg)

TT = 400
GRID = NN // TT


def _dense_body(x_lo, x_hi, g_lo, g_hi, t_lo, t_hi, deg_a, deg_b, w1, b1,
                geo_out, hsum_out):
  i = pl.program_id(0)
  x = jnp.concatenate([x_lo[0], x_hi[0]], axis=1)
  gs = jnp.concatenate([g_lo[0], g_hi[0]], axis=1)
  ts = jnp.concatenate([t_lo[0], t_hi[0]], axis=1)
  invd = 1.0 / (deg_a[0, :, 0:1] + deg_b[0, :, 0:1] + 1.0)
  geo = (gs + x) * invd
  geo_out[...] = geo
  hg = jnp.tanh(jnp.dot(geo, w1[...], preferred_element_type=jnp.float32)
                + b1[...])
  ht = jnp.tanh(jnp.dot(ts, w1[...], preferred_element_type=jnp.float32)
                + b1[...])
  s = jnp.concatenate([jnp.sum(hg, 0, keepdims=True),
                       jnp.sum(ht, 0, keepdims=True)], axis=0)

  @pl.when(i == 0)
  def _():
    hsum_out[...] = s

  @pl.when(i > 0)
  def _():
    hsum_out[...] += s


def _half(c):
  return pl.BlockSpec((1, TT, DH), lambda i, c=c: (c, i, 0))


def _dense(x2, gsum2, tsum2, deg2, w1, b1r):
  return pl.pallas_call(
      _dense_body,
      grid=(GRID,),
      in_specs=[_half(0), _half(1), _half(0), _half(1), _half(0), _half(1),
                _half(0), _half(1),
                pl.BlockSpec((DD, HH), lambda i: (0, 0)),
                pl.BlockSpec((1, HH), lambda i: (0, 0))],
      out_specs=[pl.BlockSpec((TT, DD), lambda i: (i, 0)),
                 pl.BlockSpec((2, HH), lambda i: (0, 0))],
      out_shape=[jax.ShapeDtypeStruct((NN, DD), jnp.float32),
                 jax.ShapeDtypeStruct((2, HH), jnp.float32)],
  )(x2, x2, gsum2, gsum2, tsum2, tsum2, deg2, deg2, w1, b1r)


def _combine_body(emit_q, hsum, w2r, geo, t_lo, t_hi, *outs):
  wm = jnp.sum(hsum[...] * w2r[...], axis=1) / NN
  m = jnp.max(wm)
  e = jnp.exp(wm - m)
  beta = e / jnp.sum(e)
  g = geo[...]
  lo = beta[0] * g[:, :DH] + beta[1] * t_lo[0]
  hi = beta[0] * g[:, DH:] + beta[1] * t_hi[0]
  outs[0][0] = lo
  outs[0][1] = hi
  if emit_q:
    outs[1][0] = lo[:, :QD]
    outs[1][1] = lo[:, QD:]
    outs[1][2] = hi[:, :QD]
    outs[1][3] = hi[:, QD:]


def _combine(hsum, w2r, geo, tsum2, emit_q):
  out_specs = [pl.BlockSpec((NC, TT, DH), lambda i: (0, i, 0))]
  out_shape = [jax.ShapeDtypeStruct((NC, NN, DH), jnp.float32)]
  if emit_q:
    out_specs.append(pl.BlockSpec((NQ, TT, QD), lambda i: (0, i, 0)))
    out_shape.append(jax.ShapeDtypeStruct((NQ, NN, QD), jnp.float32))
  return pl.pallas_call(
      functools.partial(_combine_body, emit_q),
      grid=(GRID,),
      in_specs=[pl.BlockSpec((2, HH), lambda i: (0, 0)),
                pl.BlockSpec((1, HH), lambda i: (0, 0)),
                pl.BlockSpec((TT, DD), lambda i: (i, 0)),
                _half(0), _half(1)],
      out_specs=out_specs,
      out_shape=out_shape,
  )(hsum, w2r, geo, tsum2, tsum2)


def kernel(loc_feat, geo_edge_index, trans_edge_index, trans_w,
           W1_0, b1_0, W2_0, W1_1, b1_1, W2_1):
  npad = EP - EE
  pad_src = jnp.arange(npad, dtype=jnp.int32) % NN
  pad_dst = NN + jnp.arange(npad, dtype=jnp.int32) % (NP - NN)

  def prep(ei):
    s = jnp.concatenate([ei[0], pad_src]).reshape(EP // KC, KC)
    d = jnp.concatenate([ei[1], pad_dst]).reshape(EP // KC, KC)
    return s, d

  src_g2, dst_g2 = prep(geo_edge_index)
  src_t2, dst_t2 = prep(trans_edge_index)
  w_t2 = jnp.concatenate(
      [trans_w, jnp.zeros((npad,), jnp.float32)]).reshape(EP // KC, KC)
  x4 = jnp.stack([loc_feat[:, q * QD:(q + 1) * QD] for q in range(NQ)])
  x2 = jnp.stack([loc_feat[:, :DH], loc_feat[:, DH:]])
  b1_0r = b1_0.reshape(1, HH)
  b1_1r = b1_1.reshape(1, HH)
  w2_0r = W2_0.reshape(1, HH)
  w2_1r = W2_1.reshape(1, HH)

  gsum2, tsum2, deg2 = _agg_deg(x4, src_g2, dst_g2, src_t2, dst_t2, w_t2)
  geo1, hsum1 = _dense(x2, gsum2, tsum2, deg2, W1_0, b1_0r)
  x2, x4 = _combine(hsum1, w2_0r, geo1, tsum2, True)

  gsum2, tsum2, _ = _agg_deg(x4, src_g2, dst_g2, src_t2, dst_t2, w_t2)
  geo2, hsum2 = _dense(x2, gsum2, tsum2, deg2, W1_1, b1_1r)
  (x2,) = _combine(hsum2, w2_1r, geo2, tsum2, False)

  return jnp.moveaxis(x2, 0, 1).reshape(NN, DD)


# speedup vs baseline: 1.3617x; 1.1159x over previous
import functools

import jax
import jax.numpy as jnp
from jax import lax
from jax.experimental import pallas as pl
from jax.experimental.pallas import tpu as pltpu
from jax.experimental.pallas import tpu_sc as plsc

NN = 10000
DD = 256
DH = 128
QD = 64
NQ = DD // QD
HH = 1024
EE = 160000
NC = 2
NS = 16
NP = 10240
RPT = NP // NS
KC = 128
EP = 163840
CPT = EP // (NS * KC)
NBUF = 4
PD = 3


@functools.cache
def _mesh():
  return plsc.VectorSubcoreMesh(
      core_axis_name="c", subcore_axis_name="s", num_cores=NC, num_subcores=NS)


def _agg_body(do_deg, x4, src_g2, dst_g2, src_t2, dst_t2, w_t2,
              gsum4, tsum4, deg_out,
              acc_sh, idx_all, dst_all, w_all,
              r0, r1, r2, r3, sg0, sg1, sg2, sg3, ss0, ss1, ss2, ss3):
  rows = (r0, r1, r2, r3)
  semg = (sg0, sg1, sg2, sg3)
  sems = (ss0, ss1, ss2, ss3)
  cid = lax.axis_index("c")
  sid = lax.axis_index("s")
  rbase = sid * RPT
  cbase = sid * CPT

  def fill(buf, val):
    v = jnp.full((16,), val, jnp.float32)

    def row(e, _):
      for j in range(QD // 16):
        buf[e, pl.ds(j * 16, 16)] = v
      return 0

    lax.fori_loop(0, KC, row, 0)

  def zero_acc():
    fill(rows[0], 0.0)
    for i in range(RPT // KC):
      pltpu.sync_copy(rows[0], acc_sh.at[pl.ds(rbase + i * KC, KC)])

  def flush(out, q):
    pltpu.sync_copy(
        acc_sh.at[pl.ds(rbase, RPT)],
        out.at[cid].at[pl.ds(rbase, RPT), pl.ds(q * QD, QD)])

  def scale_buf(buf, c):
    def group(g, _):
      w16 = w_all[c, pl.ds(g * 16, 16)]
      for lane in range(16):
        w = w16[lane]
        for j in range(QD // 16):
          buf[g * 16 + lane, pl.ds(j * 16, 16)] = (
              buf[g * 16 + lane, pl.ds(j * 16, 16)] * w)
      return 0

    lax.fori_loop(0, KC // 16, group, 0)

  def wait_gather(xq, b):
    pltpu.make_async_copy(xq.at[idx_all.at[0]], rows[b], semg[b]).wait()

  def wait_scatter(b):
    pltpu.make_async_copy(rows[b], acc_sh.at[dst_all.at[0]], sems[b]).wait()

  def quarter_pass(xq, scale):
    for b in range(PD):
      pltpu.async_copy(xq.at[idx_all.at[b]], rows[b], semg[b])

    def quad(q, _):
      for b in range(NBUF):
        c = q * NBUF + b
        wait_gather(xq, b)
        if scale:
          scale_buf(rows[b], c)
        pltpu.async_copy(rows[b], acc_sh.at[dst_all.at[c]], sems[b],
                         add=True)
        cp = c + PD
        bp = (b + PD) % NBUF

        @pl.when(cp < CPT)
        def _():
          @pl.when(cp >= NBUF)
          def _():
            wait_scatter(bp)
          pltpu.async_copy(xq.at[idx_all.at[cp]], rows[bp], semg[bp])
      return 0

    lax.fori_loop(0, CPT // NBUF, quad, 0)
    for b in range(NBUF):
      wait_scatter(b)

  def relation(src2, dst2, out, scale):
    pltpu.sync_copy(src2.at[pl.ds(cbase, CPT)], idx_all)
    pltpu.sync_copy(dst2.at[pl.ds(cbase, CPT)], dst_all)
    if scale:
      pltpu.sync_copy(w_t2.at[pl.ds(cbase, CPT)], w_all)
    for q in range(NC):
      qidx = cid * NC + q
      zero_acc()
      plsc.subcore_barrier()
      quarter_pass(x4.at[qidx], scale)
      plsc.subcore_barrier()
      flush(out, q)
      plsc.subcore_barrier()

  relation(src_g2, dst_g2, gsum4, False)

  if do_deg:
    zero_acc()
    fill(rows[1], 1.0)
    plsc.subcore_barrier()

    def dchunk(i, _):
      c = cid * (CPT // 2) + i

      @pl.when(i >= 2)
      def _():
        wait_scatter(1)

      pltpu.async_copy(rows[1], acc_sh.at[dst_all.at[c]], sems[1], add=True)
      return 0

    lax.fori_loop(0, CPT // 2, dchunk, 0)
    wait_scatter(1)
    wait_scatter(1)
    plsc.subcore_barrier()
    flush(deg_out, 0)
    plsc.subcore_barrier()

  relation(src_t2, dst_t2, tsum4, True)


def _make_agg(do_deg):
  out_type = [
      jax.ShapeDtypeStruct((NC, NP, DH), jnp.float32),
      jax.ShapeDtypeStruct((NC, NP, DH), jnp.float32),
      jax.ShapeDtypeStruct((NC, NP, DH), jnp.float32),
  ]
  if not do_deg:
    out_type = out_type[:2]
  scratch = (
      [pltpu.VMEM_SHARED((NP, QD), jnp.float32)]
      + [pltpu.VMEM((CPT, KC), jnp.int32)] * 2
      + [pltpu.VMEM((CPT, KC), jnp.float32)]
      + [pltpu.VMEM((KC, QD), jnp.float32)] * NBUF
      + [pltpu.SemaphoreType.DMA] * (2 * NBUF)
  )

  if do_deg:
    def body(x4, src_g2, dst_g2, src_t2, dst_t2, w_t2, gsum4, tsum4,
             deg_out, *scr):
      _agg_body(True, x4, src_g2, dst_g2, src_t2, dst_t2, w_t2,
                gsum4, tsum4, deg_out, *scr)
  else:
    def body(x4, src_g2, dst_g2, src_t2, dst_t2, w_t2, gsum4, tsum4, *scr):
      _agg_body(False, x4, src_g2, dst_g2, src_t2, dst_t2, w_t2,
                gsum4, tsum4, None, *scr)

  return pl.kernel(body, out_type=out_type, mesh=_mesh(),
                   scratch_types=scratch, name="sc_agg",
                   compiler_params=pltpu.CompilerParams(
                       use_tc_tiling_on_sc=False))


_agg_deg = lambda *a: _make_agg_cached(True)(*a)
_agg = lambda *a: _make_agg_cached(False)(*a)
_make_agg_cached = functools.cache(_make_agg)

TT = 400
GRID = NN // TT


def _dense_body(x_lo, x_hi, g_lo, g_hi, t_lo, t_hi, deg_a, deg_b, w1, b1,
                geo_out, hsum_out):
  i = pl.program_id(0)
  x = jnp.concatenate([x_lo[0], x_hi[0]], axis=1)
  gs = jnp.concatenate([g_lo[0], g_hi[0]], axis=1)
  ts = jnp.concatenate([t_lo[0], t_hi[0]], axis=1)
  invd = 1.0 / (deg_a[0, :, 0:1] + deg_b[0, :, 0:1] + 1.0)
  geo = (gs + x) * invd
  geo_out[...] = geo
  hg = jnp.tanh(jnp.dot(geo, w1[...], preferred_element_type=jnp.float32)
                + b1[...])
  ht = jnp.tanh(jnp.dot(ts, w1[...], preferred_element_type=jnp.float32)
                + b1[...])
  s = jnp.concatenate([jnp.sum(hg, 0, keepdims=True),
                       jnp.sum(ht, 0, keepdims=True)], axis=0)

  @pl.when(i == 0)
  def _():
    hsum_out[...] = s

  @pl.when(i > 0)
  def _():
    hsum_out[...] += s


def _half(c):
  return pl.BlockSpec((1, TT, DH), lambda i, c=c: (c, i, 0))


def _dense(x2, gsum2, tsum2, deg2, w1, b1r):
  return pl.pallas_call(
      _dense_body,
      grid=(GRID,),
      in_specs=[_half(0), _half(1), _half(0), _half(1), _half(0), _half(1),
                _half(0), _half(1),
                pl.BlockSpec((DD, HH), lambda i: (0, 0)),
                pl.BlockSpec((1, HH), lambda i: (0, 0))],
      out_specs=[pl.BlockSpec((TT, DD), lambda i: (i, 0)),
                 pl.BlockSpec((2, HH), lambda i: (0, 0))],
      out_shape=[jax.ShapeDtypeStruct((NN, DD), jnp.float32),
                 jax.ShapeDtypeStruct((2, HH), jnp.float32)],
  )(x2, x2, gsum2, gsum2, tsum2, tsum2, deg2, deg2, w1, b1r)


def _combine_body(emit_q, hsum, w2r, geo, t_lo, t_hi, *outs):
  wm = jnp.sum(hsum[...] * w2r[...], axis=1) / NN
  m = jnp.max(wm)
  e = jnp.exp(wm - m)
  beta = e / jnp.sum(e)
  g = geo[...]
  lo = beta[0] * g[:, :DH] + beta[1] * t_lo[0]
  hi = beta[0] * g[:, DH:] + beta[1] * t_hi[0]
  outs[0][0] = lo
  outs[0][1] = hi
  if emit_q:
    outs[1][0] = lo[:, :QD]
    outs[1][1] = lo[:, QD:]
    outs[1][2] = hi[:, :QD]
    outs[1][3] = hi[:, QD:]


def _combine(hsum, w2r, geo, tsum2, emit_q):
  out_specs = [pl.BlockSpec((NC, TT, DH), lambda i: (0, i, 0))]
  out_shape = [jax.ShapeDtypeStruct((NC, NN, DH), jnp.float32)]
  if emit_q:
    out_specs.append(pl.BlockSpec((NQ, TT, QD), lambda i: (0, i, 0)))
    out_shape.append(jax.ShapeDtypeStruct((NQ, NN, QD), jnp.float32))
  return pl.pallas_call(
      functools.partial(_combine_body, emit_q),
      grid=(GRID,),
      in_specs=[pl.BlockSpec((2, HH), lambda i: (0, 0)),
                pl.BlockSpec((1, HH), lambda i: (0, 0)),
                pl.BlockSpec((TT, DD), lambda i: (i, 0)),
                _half(0), _half(1)],
      out_specs=out_specs,
      out_shape=out_shape,
  )(hsum, w2r, geo, tsum2, tsum2)


def kernel(loc_feat, geo_edge_index, trans_edge_index, trans_w,
           W1_0, b1_0, W2_0, W1_1, b1_1, W2_1):
  npad = EP - EE
  pad_src = jnp.arange(npad, dtype=jnp.int32) % NN
  pad_dst = NN + jnp.arange(npad, dtype=jnp.int32) % (NP - NN)

  def prep(ei):
    s = jnp.concatenate([ei[0], pad_src]).reshape(EP // KC, KC)
    d = jnp.concatenate([ei[1], pad_dst]).reshape(EP // KC, KC)
    return s, d

  src_g2, dst_g2 = prep(geo_edge_index)
  src_t2, dst_t2 = prep(trans_edge_index)
  w_t2 = jnp.concatenate(
      [trans_w, jnp.zeros((npad,), jnp.float32)]).reshape(EP // KC, KC)
  x4 = jnp.stack([loc_feat[:, q * QD:(q + 1) * QD] for q in range(NQ)])
  x2 = jnp.stack([loc_feat[:, :DH], loc_feat[:, DH:]])
  b1_0r = b1_0.reshape(1, HH)
  b1_1r = b1_1.reshape(1, HH)
  w2_0r = W2_0.reshape(1, HH)
  w2_1r = W2_1.reshape(1, HH)

  gsum2, tsum2, deg2 = _agg_deg(x4, src_g2, dst_g2, src_t2, dst_t2, w_t2)
  geo1, hsum1 = _dense(x2, gsum2, tsum2, deg2, W1_0, b1_0r)
  x2, x4 = _combine(hsum1, w2_0r, geo1, tsum2, True)

  gsum2, tsum2, _ = _agg_deg(x4, src_g2, dst_g2, src_t2, dst_t2, w_t2)
  geo2, hsum2 = _dense(x2, gsum2, tsum2, deg2, W1_1, b1_1r)
  (x2,) = _combine(hsum2, w2_1r, geo2, tsum2, False)

  return jnp.moveaxis(x2, 0, 1).reshape(NN, DD)


# speedup vs baseline: 1.3681x; 1.0048x over previous
import functools

import jax
import jax.numpy as jnp
from jax import lax
from jax.experimental import pallas as pl
from jax.experimental.pallas import tpu as pltpu
from jax.experimental.pallas import tpu_sc as plsc

NN = 10000
DD = 256
DH = 128
QD = 64
NQ = DD // QD
HH = 1024
EE = 160000
NC = 2
NS = 16
NP = 10240
RPT = NP // NS
KC = 128
EP = 163840
CPT = EP // (NS * KC)
NBUF = 4
PD = 3


@functools.cache
def _mesh():
  return plsc.VectorSubcoreMesh(
      core_axis_name="c", subcore_axis_name="s", num_cores=NC, num_subcores=NS)


def _agg_body(do_deg, x4, src_g2, dst_g2, src_t2, dst_t2, w_t2,
              gsum4, tsum4, deg_out,
              acc_sh, idx_all, dst_all, w_all,
              r0, r1, r2, r3, sg0, sg1, sg2, sg3, ss0, ss1, ss2, ss3):
  rows = (r0, r1, r2, r3)
  semg = (sg0, sg1, sg2, sg3)
  sems = (ss0, ss1, ss2, ss3)
  cid = lax.axis_index("c")
  sid = lax.axis_index("s")
  rbase = sid * RPT
  cbase = sid * CPT

  def fill(buf, val):
    v = jnp.full((16,), val, jnp.float32)

    def row(e, _):
      for j in range(QD // 16):
        buf[e, pl.ds(j * 16, 16)] = v
      return 0

    lax.fori_loop(0, KC, row, 0)

  def zero_acc():
    fill(rows[0], 0.0)
    for i in range(RPT // KC):
      pltpu.sync_copy(rows[0], acc_sh.at[pl.ds(rbase + i * KC, KC)])

  def flush(out, q):
    pltpu.sync_copy(
        acc_sh.at[pl.ds(rbase, RPT)],
        out.at[cid].at[pl.ds(rbase, RPT), pl.ds(q * QD, QD)])

  def scale_buf(buf, c):
    def group(g, _):
      w16 = w_all[c, pl.ds(g * 16, 16)]
      for lane in range(16):
        w = w16[lane]
        for j in range(QD // 16):
          buf[g * 16 + lane, pl.ds(j * 16, 16)] = (
              buf[g * 16 + lane, pl.ds(j * 16, 16)] * w)
      return 0

    lax.fori_loop(0, KC // 16, group, 0)

  def wait_gather(xq, b):
    pltpu.make_async_copy(xq.at[idx_all.at[0]], rows[b], semg[b]).wait()

  def wait_scatter(b):
    pltpu.make_async_copy(rows[b], acc_sh.at[dst_all.at[0]], sems[b]).wait()

  def quarter_pass(xq, scale):
    for b in range(PD):
      pltpu.async_copy(xq.at[idx_all.at[b]], rows[b], semg[b])

    def quad(q, _):
      for b in range(NBUF):
        c = q * NBUF + b
        wait_gather(xq, b)
        if scale:
          scale_buf(rows[b], c)
        pltpu.async_copy(rows[b], acc_sh.at[dst_all.at[c]], sems[b],
                         add=True)
        cp = c + PD
        bp = (b + PD) % NBUF

        @pl.when(cp < CPT)
        def _():
          @pl.when(cp >= NBUF)
          def _():
            wait_scatter(bp)
          pltpu.async_copy(xq.at[idx_all.at[cp]], rows[bp], semg[bp])
      return 0

    lax.fori_loop(0, CPT // NBUF, quad, 0)
    for b in range(NBUF):
      wait_scatter(b)

  def relation(src2, dst2, out, scale):
    pltpu.sync_copy(src2.at[pl.ds(cbase, CPT)], idx_all)
    pltpu.sync_copy(dst2.at[pl.ds(cbase, CPT)], dst_all)
    if scale:
      pltpu.sync_copy(w_t2.at[pl.ds(cbase, CPT)], w_all)
    for q in range(NC):
      qidx = cid * NC + q
      zero_acc()
      plsc.subcore_barrier()
      quarter_pass(x4.at[qidx], scale)
      plsc.subcore_barrier()
      flush(out, q)
      plsc.subcore_barrier()

  relation(src_g2, dst_g2, gsum4, False)

  if do_deg:
    zero_acc()
    fill(rows[1], 1.0)
    plsc.subcore_barrier()

    def dchunk(i, _):
      c = cid * (CPT // 2) + i

      @pl.when(i >= 2)
      def _():
        wait_scatter(1)

      pltpu.async_copy(rows[1], acc_sh.at[dst_all.at[c]], sems[1], add=True)
      return 0

    lax.fori_loop(0, CPT // 2, dchunk, 0)
    wait_scatter(1)
    wait_scatter(1)
    plsc.subcore_barrier()
    flush(deg_out, 0)
    plsc.subcore_barrier()

  relation(src_t2, dst_t2, tsum4, True)


def _make_agg(do_deg):
  out_type = [
      jax.ShapeDtypeStruct((NC, NP, DH), jnp.float32),
      jax.ShapeDtypeStruct((NC, NP, DH), jnp.float32),
      jax.ShapeDtypeStruct((NC, NP, DH), jnp.float32),
  ]
  if not do_deg:
    out_type = out_type[:2]
  scratch = (
      [pltpu.VMEM_SHARED((NP, QD), jnp.float32)]
      + [pltpu.VMEM((CPT, KC), jnp.int32)] * 2
      + [pltpu.VMEM((CPT, KC), jnp.float32)]
      + [pltpu.VMEM((KC, QD), jnp.float32)] * NBUF
      + [pltpu.SemaphoreType.DMA] * (2 * NBUF)
  )

  if do_deg:
    def body(x4, src_g2, dst_g2, src_t2, dst_t2, w_t2, gsum4, tsum4,
             deg_out, *scr):
      _agg_body(True, x4, src_g2, dst_g2, src_t2, dst_t2, w_t2,
                gsum4, tsum4, deg_out, *scr)
  else:
    def body(x4, src_g2, dst_g2, src_t2, dst_t2, w_t2, gsum4, tsum4, *scr):
      _agg_body(False, x4, src_g2, dst_g2, src_t2, dst_t2, w_t2,
                gsum4, tsum4, None, *scr)

  return pl.kernel(body, out_type=out_type, mesh=_mesh(),
                   scratch_types=scratch, name="sc_agg",
                   compiler_params=pltpu.CompilerParams(
                       use_tc_tiling_on_sc=False))


_agg_deg = lambda *a: _make_agg_cached(True)(*a)
_agg = lambda *a: _make_agg_cached(False)(*a)
_make_agg_cached = functools.cache(_make_agg)

TT = 400
GRID = NN // TT


def _dense_body(x_lo, x_hi, g_lo, g_hi, t_lo, t_hi, deg_a, deg_b, w1, b1,
                geo_out, hsum_out):
  i = pl.program_id(0)
  x = jnp.concatenate([x_lo[0], x_hi[0]], axis=1)
  gs = jnp.concatenate([g_lo[0], g_hi[0]], axis=1)
  ts = jnp.concatenate([t_lo[0], t_hi[0]], axis=1)
  invd = 1.0 / (deg_a[0, :, 0:1] + deg_b[0, :, 0:1] + 1.0)
  geo = (gs + x) * invd
  geo_out[...] = geo
  w1b = w1[...]
  hg = jnp.tanh(jnp.dot(geo.astype(jnp.bfloat16), w1b,
                        preferred_element_type=jnp.float32) + b1[...])
  ht = jnp.tanh(jnp.dot(ts.astype(jnp.bfloat16), w1b,
                        preferred_element_type=jnp.float32) + b1[...])
  s = jnp.concatenate([jnp.sum(hg, 0, keepdims=True),
                       jnp.sum(ht, 0, keepdims=True)], axis=0)

  @pl.when(i == 0)
  def _():
    hsum_out[...] = s

  @pl.when(i > 0)
  def _():
    hsum_out[...] += s


def _half(c):
  return pl.BlockSpec((1, TT, DH), lambda i, c=c: (c, i, 0))


def _dense(x2, gsum2, tsum2, deg2, w1, b1r):
  return pl.pallas_call(
      _dense_body,
      grid=(GRID,),
      in_specs=[_half(0), _half(1), _half(0), _half(1), _half(0), _half(1),
                _half(0), _half(1),
                pl.BlockSpec((DD, HH), lambda i: (0, 0)),
                pl.BlockSpec((1, HH), lambda i: (0, 0))],
      out_specs=[pl.BlockSpec((TT, DD), lambda i: (i, 0)),
                 pl.BlockSpec((2, HH), lambda i: (0, 0))],
      out_shape=[jax.ShapeDtypeStruct((NN, DD), jnp.float32),
                 jax.ShapeDtypeStruct((2, HH), jnp.float32)],
  )(x2, x2, gsum2, gsum2, tsum2, tsum2, deg2, deg2, w1, b1r)


def _combine_body(emit_q, hsum, w2r, geo, t_lo, t_hi, *outs):
  wm = jnp.sum(hsum[...] * w2r[...], axis=1) / NN
  m = jnp.max(wm)
  e = jnp.exp(wm - m)
  beta = e / jnp.sum(e)
  g = geo[...]
  lo = beta[0] * g[:, :DH] + beta[1] * t_lo[0]
  hi = beta[0] * g[:, DH:] + beta[1] * t_hi[0]
  outs[0][0] = lo
  outs[0][1] = hi
  if emit_q:
    outs[1][0] = lo[:, :QD]
    outs[1][1] = lo[:, QD:]
    outs[1][2] = hi[:, :QD]
    outs[1][3] = hi[:, QD:]


def _combine(hsum, w2r, geo, tsum2, emit_q):
  out_specs = [pl.BlockSpec((NC, TT, DH), lambda i: (0, i, 0))]
  out_shape = [jax.ShapeDtypeStruct((NC, NN, DH), jnp.float32)]
  if emit_q:
    out_specs.append(pl.BlockSpec((NQ, TT, QD), lambda i: (0, i, 0)))
    out_shape.append(jax.ShapeDtypeStruct((NQ, NN, QD), jnp.float32))
  return pl.pallas_call(
      functools.partial(_combine_body, emit_q),
      grid=(GRID,),
      in_specs=[pl.BlockSpec((2, HH), lambda i: (0, 0)),
                pl.BlockSpec((1, HH), lambda i: (0, 0)),
                pl.BlockSpec((TT, DD), lambda i: (i, 0)),
                _half(0), _half(1)],
      out_specs=out_specs,
      out_shape=out_shape,
  )(hsum, w2r, geo, tsum2, tsum2)


def kernel(loc_feat, geo_edge_index, trans_edge_index, trans_w,
           W1_0, b1_0, W2_0, W1_1, b1_1, W2_1):
  npad = EP - EE
  pad_src = jnp.arange(npad, dtype=jnp.int32) % NN
  pad_dst = NN + jnp.arange(npad, dtype=jnp.int32) % (NP - NN)

  def prep(ei):
    s = jnp.concatenate([ei[0], pad_src]).reshape(EP // KC, KC)
    d = jnp.concatenate([ei[1], pad_dst]).reshape(EP // KC, KC)
    return s, d

  src_g2, dst_g2 = prep(geo_edge_index)
  src_t2, dst_t2 = prep(trans_edge_index)
  w_t2 = jnp.concatenate(
      [trans_w, jnp.zeros((npad,), jnp.float32)]).reshape(EP // KC, KC)
  x4 = jnp.stack([loc_feat[:, q * QD:(q + 1) * QD] for q in range(NQ)])
  x2 = jnp.stack([loc_feat[:, :DH], loc_feat[:, DH:]])
  b1_0r = b1_0.reshape(1, HH)
  b1_1r = b1_1.reshape(1, HH)
  w2_0r = W2_0.reshape(1, HH)
  w2_1r = W2_1.reshape(1, HH)

  w1_0b = W1_0.astype(jnp.bfloat16)
  w1_1b = W1_1.astype(jnp.bfloat16)

  gsum2, tsum2, deg2 = _agg_deg(x4, src_g2, dst_g2, src_t2, dst_t2, w_t2)
  geo1, hsum1 = _dense(x2, gsum2, tsum2, deg2, w1_0b, b1_0r)
  x2, x4 = _combine(hsum1, w2_0r, geo1, tsum2, True)

  gsum2, tsum2, _ = _agg_deg(x4, src_g2, dst_g2, src_t2, dst_t2, w_t2)
  geo2, hsum2 = _dense(x2, gsum2, tsum2, deg2, w1_1b, b1_1r)
  (x2,) = _combine(hsum2, w2_1r, geo2, tsum2, False)

  return jnp.moveaxis(x2, 0, 1).reshape(NN, DD)


# speedup vs baseline: 1.3846x; 1.0121x over previous
import functools

import jax
import jax.numpy as jnp
from jax import lax
from jax.experimental import pallas as pl
from jax.experimental.pallas import tpu as pltpu
from jax.experimental.pallas import tpu_sc as plsc

NN = 10000
DD = 256
DH = 128
QD = 64
NQ = DD // QD
HH = 1024
EE = 160000
NC = 2
NS = 16
NP = 10240
RPT = NP // NS
KC = 128
EP = 163840
CPT = EP // (NS * KC)
NBUF = 5
PD = 3


@functools.cache
def _mesh():
  return plsc.VectorSubcoreMesh(
      core_axis_name="c", subcore_axis_name="s", num_cores=NC, num_subcores=NS)


def _agg_body(do_deg, x4, src_g2, dst_g2, src_t2, dst_t2, w_t2,
              gsum4, tsum4, deg_out,
              acc_sh, idx_all, dst_all, w_all, *bufs):
  rows = bufs[:NBUF]
  semg = bufs[NBUF:2 * NBUF]
  sems = bufs[2 * NBUF:3 * NBUF]
  cid = lax.axis_index("c")
  sid = lax.axis_index("s")
  rbase = sid * RPT
  cbase = sid * CPT

  def fill(buf, val):
    v = jnp.full((16,), val, jnp.float32)

    def row(e, _):
      for j in range(QD // 16):
        buf[e, pl.ds(j * 16, 16)] = v
      return 0

    lax.fori_loop(0, KC, row, 0)

  def zero_acc():
    fill(rows[0], 0.0)
    for i in range(RPT // KC):
      pltpu.sync_copy(rows[0], acc_sh.at[pl.ds(rbase + i * KC, KC)])

  def flush(out, q):
    pltpu.sync_copy(
        acc_sh.at[pl.ds(rbase, RPT)],
        out.at[cid].at[pl.ds(rbase, RPT), pl.ds(q * QD, QD)])

  def scale_buf(buf, c):
    def group(g, _):
      w16 = w_all[c, pl.ds(g * 16, 16)]
      for lane in range(16):
        w = w16[lane]
        for j in range(QD // 16):
          buf[g * 16 + lane, pl.ds(j * 16, 16)] = (
              buf[g * 16 + lane, pl.ds(j * 16, 16)] * w)
      return 0

    lax.fori_loop(0, KC // 16, group, 0)

  def wait_gather(xq, b):
    pltpu.make_async_copy(xq.at[idx_all.at[0]], rows[b], semg[b]).wait()

  def wait_scatter(b):
    pltpu.make_async_copy(rows[b], acc_sh.at[dst_all.at[0]], sems[b]).wait()

  def quarter_pass(xq, scale):
    for b in range(PD):
      pltpu.async_copy(xq.at[idx_all.at[b]], rows[b], semg[b])

    def quad(q, _):
      for b in range(NBUF):
        c = q * NBUF + b
        wait_gather(xq, b)
        if scale:
          scale_buf(rows[b], c)
        pltpu.async_copy(rows[b], acc_sh.at[dst_all.at[c]], sems[b],
                         add=True)
        cp = c + PD
        bp = (b + PD) % NBUF

        @pl.when(cp < CPT)
        def _():
          @pl.when(cp >= NBUF)
          def _():
            wait_scatter(bp)
          pltpu.async_copy(xq.at[idx_all.at[cp]], rows[bp], semg[bp])
      return 0

    lax.fori_loop(0, CPT // NBUF, quad, 0)
    for b in range(NBUF):
      wait_scatter(b)

  def relation(src2, dst2, out, scale):
    pltpu.sync_copy(src2.at[pl.ds(cbase, CPT)], idx_all)
    pltpu.sync_copy(dst2.at[pl.ds(cbase, CPT)], dst_all)
    if scale:
      pltpu.sync_copy(w_t2.at[pl.ds(cbase, CPT)], w_all)
    for q in range(NC):
      qidx = cid * NC + q
      zero_acc()
      plsc.subcore_barrier()
      quarter_pass(x4.at[qidx], scale)
      plsc.subcore_barrier()
      flush(out, q)
      plsc.subcore_barrier()

  relation(src_g2, dst_g2, gsum4, False)

  if do_deg:
    zero_acc()
    fill(rows[1], 1.0)
    plsc.subcore_barrier()

    def dchunk(i, _):
      c = cid * (CPT // 2) + i

      @pl.when(i >= 2)
      def _():
        wait_scatter(1)

      pltpu.async_copy(rows[1], acc_sh.at[dst_all.at[c]], sems[1], add=True)
      return 0

    lax.fori_loop(0, CPT // 2, dchunk, 0)
    wait_scatter(1)
    wait_scatter(1)
    plsc.subcore_barrier()
    flush(deg_out, 0)
    plsc.subcore_barrier()

  relation(src_t2, dst_t2, tsum4, True)


def _make_agg(do_deg):
  out_type = [
      jax.ShapeDtypeStruct((NC, NP, DH), jnp.float32),
      jax.ShapeDtypeStruct((NC, NP, DH), jnp.float32),
      jax.ShapeDtypeStruct((NC, NP, DH), jnp.float32),
  ]
  if not do_deg:
    out_type = out_type[:2]
  scratch = (
      [pltpu.VMEM_SHARED((NP, QD), jnp.float32)]
      + [pltpu.VMEM((CPT, KC), jnp.int32)] * 2
      + [pltpu.VMEM((CPT, KC), jnp.float32)]
      + [pltpu.VMEM((KC, QD), jnp.float32)] * NBUF
      + [pltpu.SemaphoreType.DMA] * (2 * NBUF)
  )

  if do_deg:
    def body(x4, src_g2, dst_g2, src_t2, dst_t2, w_t2, gsum4, tsum4,
             deg_out, *scr):
      _agg_body(True, x4, src_g2, dst_g2, src_t2, dst_t2, w_t2,
                gsum4, tsum4, deg_out, *scr)
  else:
    def body(x4, src_g2, dst_g2, src_t2, dst_t2, w_t2, gsum4, tsum4, *scr):
      _agg_body(False, x4, src_g2, dst_g2, src_t2, dst_t2, w_t2,
                gsum4, tsum4, None, *scr)

  return pl.kernel(body, out_type=out_type, mesh=_mesh(),
                   scratch_types=scratch, name="sc_agg",
                   compiler_params=pltpu.CompilerParams(
                       use_tc_tiling_on_sc=False))


_agg_deg = lambda *a: _make_agg_cached(True)(*a)
_agg = lambda *a: _make_agg_cached(False)(*a)
_make_agg_cached = functools.cache(_make_agg)

TT = 400
GRID = NN // TT


def _dense_body(x_lo, x_hi, g_lo, g_hi, t_lo, t_hi, deg_a, deg_b, w1, b1,
                geo_out, hsum_out):
  i = pl.program_id(0)
  x = jnp.concatenate([x_lo[0], x_hi[0]], axis=1)
  gs = jnp.concatenate([g_lo[0], g_hi[0]], axis=1)
  ts = jnp.concatenate([t_lo[0], t_hi[0]], axis=1)
  invd = 1.0 / (deg_a[0, :, 0:1] + deg_b[0, :, 0:1] + 1.0)
  geo = (gs + x) * invd
  geo_out[...] = geo
  w1b = w1[...]
  hg = jnp.tanh(jnp.dot(geo.astype(jnp.bfloat16), w1b,
                        preferred_element_type=jnp.float32) + b1[...])
  ht = jnp.tanh(jnp.dot(ts.astype(jnp.bfloat16), w1b,
                        preferred_element_type=jnp.float32) + b1[...])
  s = jnp.concatenate([jnp.sum(hg, 0, keepdims=True),
                       jnp.sum(ht, 0, keepdims=True)], axis=0)

  @pl.when(i == 0)
  def _():
    hsum_out[...] = s

  @pl.when(i > 0)
  def _():
    hsum_out[...] += s


def _half(c):
  return pl.BlockSpec((1, TT, DH), lambda i, c=c: (c, i, 0))


def _dense(x2, gsum2, tsum2, deg2, w1, b1r):
  return pl.pallas_call(
      _dense_body,
      grid=(GRID,),
      in_specs=[_half(0), _half(1), _half(0), _half(1), _half(0), _half(1),
                _half(0), _half(1),
                pl.BlockSpec((DD, HH), lambda i: (0, 0)),
                pl.BlockSpec((1, HH), lambda i: (0, 0))],
      out_specs=[pl.BlockSpec((TT, DD), lambda i: (i, 0)),
                 pl.BlockSpec((2, HH), lambda i: (0, 0))],
      out_shape=[jax.ShapeDtypeStruct((NN, DD), jnp.float32),
                 jax.ShapeDtypeStruct((2, HH), jnp.float32)],
  )(x2, x2, gsum2, gsum2, tsum2, tsum2, deg2, deg2, w1, b1r)


def _combine_body(emit_q, hsum, w2r, geo, t_lo, t_hi, *outs):
  wm = jnp.sum(hsum[...] * w2r[...], axis=1) / NN
  m = jnp.max(wm)
  e = jnp.exp(wm - m)
  beta = e / jnp.sum(e)
  g = geo[...]
  lo = beta[0] * g[:, :DH] + beta[1] * t_lo[0]
  hi = beta[0] * g[:, DH:] + beta[1] * t_hi[0]
  outs[0][0] = lo
  outs[0][1] = hi
  if emit_q:
    outs[1][0] = lo[:, :QD]
    outs[1][1] = lo[:, QD:]
    outs[1][2] = hi[:, :QD]
    outs[1][3] = hi[:, QD:]


def _combine(hsum, w2r, geo, tsum2, emit_q):
  out_specs = [pl.BlockSpec((NC, TT, DH), lambda i: (0, i, 0))]
  out_shape = [jax.ShapeDtypeStruct((NC, NN, DH), jnp.float32)]
  if emit_q:
    out_specs.append(pl.BlockSpec((NQ, TT, QD), lambda i: (0, i, 0)))
    out_shape.append(jax.ShapeDtypeStruct((NQ, NN, QD), jnp.float32))
  return pl.pallas_call(
      functools.partial(_combine_body, emit_q),
      grid=(GRID,),
      in_specs=[pl.BlockSpec((2, HH), lambda i: (0, 0)),
                pl.BlockSpec((1, HH), lambda i: (0, 0)),
                pl.BlockSpec((TT, DD), lambda i: (i, 0)),
                _half(0), _half(1)],
      out_specs=out_specs,
      out_shape=out_shape,
  )(hsum, w2r, geo, tsum2, tsum2)


def kernel(loc_feat, geo_edge_index, trans_edge_index, trans_w,
           W1_0, b1_0, W2_0, W1_1, b1_1, W2_1):
  npad = EP - EE
  pad_src = jnp.arange(npad, dtype=jnp.int32) % NN
  pad_dst = NN + jnp.arange(npad, dtype=jnp.int32) % (NP - NN)

  def prep(ei):
    s = jnp.concatenate([ei[0], pad_src]).reshape(EP // KC, KC)
    d = jnp.concatenate([ei[1], pad_dst]).reshape(EP // KC, KC)
    return s, d

  src_g2, dst_g2 = prep(geo_edge_index)
  src_t2, dst_t2 = prep(trans_edge_index)
  w_t2 = jnp.concatenate(
      [trans_w, jnp.zeros((npad,), jnp.float32)]).reshape(EP // KC, KC)
  x4 = jnp.stack([loc_feat[:, q * QD:(q + 1) * QD] for q in range(NQ)])
  x2 = jnp.stack([loc_feat[:, :DH], loc_feat[:, DH:]])
  b1_0r = b1_0.reshape(1, HH)
  b1_1r = b1_1.reshape(1, HH)
  w2_0r = W2_0.reshape(1, HH)
  w2_1r = W2_1.reshape(1, HH)

  w1_0b = W1_0.astype(jnp.bfloat16)
  w1_1b = W1_1.astype(jnp.bfloat16)

  gsum2, tsum2, deg2 = _agg_deg(x4, src_g2, dst_g2, src_t2, dst_t2, w_t2)
  geo1, hsum1 = _dense(x2, gsum2, tsum2, deg2, w1_0b, b1_0r)
  x2, x4 = _combine(hsum1, w2_0r, geo1, tsum2, True)

  gsum2, tsum2, _ = _agg_deg(x4, src_g2, dst_g2, src_t2, dst_t2, w_t2)
  geo2, hsum2 = _dense(x2, gsum2, tsum2, deg2, w1_1b, b1_1r)
  (x2,) = _combine(hsum2, w2_1r, geo2, tsum2, False)

  return jnp.moveaxis(x2, 0, 1).reshape(NN, DD)


# speedup vs baseline: 1.4221x; 1.0270x over previous
import functools

import jax
import jax.numpy as jnp
from jax import lax
from jax.experimental import pallas as pl
from jax.experimental.pallas import tpu as pltpu
from jax.experimental.pallas import tpu_sc as plsc

NN = 10000
DD = 256
DH = 128
QD = 64
NQ = DD // QD
HH = 1024
EE = 160000
NC = 2
NS = 16
NP = 10240
RPT = NP // NS
KC = 128
EP = 163840
CPT = EP // (NS * KC)
NBUF = 5
PD = 3


@functools.cache
def _mesh():
  return plsc.VectorSubcoreMesh(
      core_axis_name="c", subcore_axis_name="s", num_cores=NC, num_subcores=NS)


def _agg_body(x4, src_g2, dst_g2, src_t2, dst_t2, w_t2, dflag,
              gsum4, tsum4, deg_out,
              acc_sh, idx_all, dst_all, w_all, flag_v, *bufs):
  rows = bufs[:NBUF]
  semg = bufs[NBUF:2 * NBUF]
  sems = bufs[2 * NBUF:3 * NBUF]
  cid = lax.axis_index("c")
  sid = lax.axis_index("s")
  rbase = sid * RPT
  cbase = sid * CPT

  def fill(buf, val):
    v = jnp.full((16,), val, jnp.float32)

    def row(e, _):
      for j in range(QD // 16):
        buf[e, pl.ds(j * 16, 16)] = v
      return 0

    lax.fori_loop(0, KC, row, 0)

  def zero_acc():
    fill(rows[0], 0.0)
    for i in range(RPT // KC):
      pltpu.sync_copy(rows[0], acc_sh.at[pl.ds(rbase + i * KC, KC)])

  def flush(out, q):
    pltpu.sync_copy(
        acc_sh.at[pl.ds(rbase, RPT)],
        out.at[cid].at[pl.ds(rbase, RPT), pl.ds(q * QD, QD)])

  def scale_buf(buf, c):
    def group(g, _):
      w16 = w_all[c, pl.ds(g * 16, 16)]
      for lane in range(16):
        w = w16[lane]
        for j in range(QD // 16):
          buf[g * 16 + lane, pl.ds(j * 16, 16)] = (
              buf[g * 16 + lane, pl.ds(j * 16, 16)] * w)
      return 0

    lax.fori_loop(0, KC // 16, group, 0)

  def wait_gather(xq, b):
    pltpu.make_async_copy(xq.at[idx_all.at[0]], rows[b], semg[b]).wait()

  def wait_scatter(b):
    pltpu.make_async_copy(rows[b], acc_sh.at[dst_all.at[0]], sems[b]).wait()

  def quarter_pass(xq, scale):
    for b in range(PD):
      pltpu.async_copy(xq.at[idx_all.at[b]], rows[b], semg[b])

    def quad(q, _):
      for b in range(NBUF):
        c = q * NBUF + b
        wait_gather(xq, b)
        if scale:
          scale_buf(rows[b], c)
        pltpu.async_copy(rows[b], acc_sh.at[dst_all.at[c]], sems[b],
                         add=True)
        cp = c + PD
        bp = (b + PD) % NBUF

        @pl.when(cp < CPT)
        def _():
          @pl.when(cp >= NBUF)
          def _():
            wait_scatter(bp)
          pltpu.async_copy(xq.at[idx_all.at[cp]], rows[bp], semg[bp])
      return 0

    lax.fori_loop(0, CPT // NBUF, quad, 0)
    for b in range(NBUF):
      wait_scatter(b)

  def relation(src2, dst2, out, scale):
    pltpu.sync_copy(src2.at[pl.ds(cbase, CPT)], idx_all)
    pltpu.sync_copy(dst2.at[pl.ds(cbase, CPT)], dst_all)
    if scale:
      pltpu.sync_copy(w_t2.at[pl.ds(cbase, CPT)], w_all)
    for q in range(NC):
      qidx = cid * NC + q
      zero_acc()
      plsc.subcore_barrier()
      quarter_pass(x4.at[qidx], scale)
      plsc.subcore_barrier()
      flush(out, q)
      plsc.subcore_barrier()

  relation(src_g2, dst_g2, gsum4, False)

  pltpu.sync_copy(dflag, flag_v)
  do_deg = jnp.sum(flag_v[...]) > 0

  @pl.when(do_deg)
  def _():
    zero_acc()
    fill(rows[1], 1.0)

  plsc.subcore_barrier()

  @pl.when(do_deg)
  def _():
    def dchunk(i, _):
      c = cid * (CPT // 2) + i

      @pl.when(i >= 2)
      def _():
        wait_scatter(1)

      pltpu.async_copy(rows[1], acc_sh.at[dst_all.at[c]], sems[1], add=True)
      return 0

    lax.fori_loop(0, CPT // 2, dchunk, 0)
    wait_scatter(1)
    wait_scatter(1)

  plsc.subcore_barrier()

  @pl.when(do_deg)
  def _():
    flush(deg_out, 0)

  plsc.subcore_barrier()

  relation(src_t2, dst_t2, tsum4, True)


def _make_agg():
  out_type = [
      jax.ShapeDtypeStruct((NC, NP, DH), jnp.float32),
      jax.ShapeDtypeStruct((NC, NP, DH), jnp.float32),
      jax.ShapeDtypeStruct((NC, NP, DH), jnp.float32),
  ]
  scratch = (
      [pltpu.VMEM_SHARED((NP, QD), jnp.float32)]
      + [pltpu.VMEM((CPT, KC), jnp.int32)] * 2
      + [pltpu.VMEM((CPT, KC), jnp.float32)]
      + [pltpu.VMEM((16,), jnp.int32)]
      + [pltpu.VMEM((KC, QD), jnp.float32)] * NBUF
      + [pltpu.SemaphoreType.DMA] * (2 * NBUF)
  )
  return pl.kernel(_agg_body, out_type=out_type, mesh=_mesh(),
                   scratch_types=scratch, name="sc_agg",
                   compiler_params=pltpu.CompilerParams(
                       use_tc_tiling_on_sc=False,
                       needs_layout_passes=False))


_agg = lambda *a: _make_agg_cached()(*a)
_make_agg_cached = functools.cache(_make_agg)

TT = 400
GRID = NN // TT


def _dense_body(x_lo, x_hi, g_lo, g_hi, t_lo, t_hi, deg_a, deg_b, w1, b1,
                geo_out, hsum_out):
  i = pl.program_id(0)
  x = jnp.concatenate([x_lo[0], x_hi[0]], axis=1)
  gs = jnp.concatenate([g_lo[0], g_hi[0]], axis=1)
  ts = jnp.concatenate([t_lo[0], t_hi[0]], axis=1)
  invd = 1.0 / (deg_a[0, :, 0:1] + deg_b[0, :, 0:1] + 1.0)
  geo = (gs + x) * invd
  geo_out[...] = geo
  w1b = w1[...]
  hg = jnp.tanh(jnp.dot(geo.astype(jnp.bfloat16), w1b,
                        preferred_element_type=jnp.float32) + b1[...])
  ht = jnp.tanh(jnp.dot(ts.astype(jnp.bfloat16), w1b,
                        preferred_element_type=jnp.float32) + b1[...])
  s = jnp.concatenate([jnp.sum(hg, 0, keepdims=True),
                       jnp.sum(ht, 0, keepdims=True)], axis=0)

  @pl.when(i == 0)
  def _():
    hsum_out[...] = s

  @pl.when(i > 0)
  def _():
    hsum_out[...] += s


def _half(c):
  return pl.BlockSpec((1, TT, DH), lambda i, c=c: (c, i, 0))


def _dense(x2, gsum2, tsum2, deg2, w1, b1r):
  return pl.pallas_call(
      _dense_body,
      grid=(GRID,),
      in_specs=[_half(0), _half(1), _half(0), _half(1), _half(0), _half(1),
                _half(0), _half(1),
                pl.BlockSpec((DD, HH), lambda i: (0, 0)),
                pl.BlockSpec((1, HH), lambda i: (0, 0))],
      out_specs=[pl.BlockSpec((TT, DD), lambda i: (i, 0)),
                 pl.BlockSpec((2, HH), lambda i: (0, 0))],
      out_shape=[jax.ShapeDtypeStruct((NN, DD), jnp.float32),
                 jax.ShapeDtypeStruct((2, HH), jnp.float32)],
  )(x2, x2, gsum2, gsum2, tsum2, tsum2, deg2, deg2, w1, b1r)


def _combine_body(emit_q, hsum, w2r, geo, t_lo, t_hi, *outs):
  wm = jnp.sum(hsum[...] * w2r[...], axis=1) / NN
  m = jnp.max(wm)
  e = jnp.exp(wm - m)
  beta = e / jnp.sum(e)
  g = geo[...]
  lo = beta[0] * g[:, :DH] + beta[1] * t_lo[0]
  hi = beta[0] * g[:, DH:] + beta[1] * t_hi[0]
  outs[0][0] = lo
  outs[0][1] = hi
  if emit_q:
    outs[1][0] = lo[:, :QD]
    outs[1][1] = lo[:, QD:]
    outs[1][2] = hi[:, :QD]
    outs[1][3] = hi[:, QD:]


def _combine(hsum, w2r, geo, tsum2, emit_q):
  out_specs = [pl.BlockSpec((NC, TT, DH), lambda i: (0, i, 0))]
  out_shape = [jax.ShapeDtypeStruct((NC, NN, DH), jnp.float32)]
  if emit_q:
    out_specs.append(pl.BlockSpec((NQ, TT, QD), lambda i: (0, i, 0)))
    out_shape.append(jax.ShapeDtypeStruct((NQ, NN, QD), jnp.float32))
  return pl.pallas_call(
      functools.partial(_combine_body, emit_q),
      grid=(GRID,),
      in_specs=[pl.BlockSpec((2, HH), lambda i: (0, 0)),
                pl.BlockSpec((1, HH), lambda i: (0, 0)),
                pl.BlockSpec((TT, DD), lambda i: (i, 0)),
                _half(0), _half(1)],
      out_specs=out_specs,
      out_shape=out_shape,
  )(hsum, w2r, geo, tsum2, tsum2)


def kernel(loc_feat, geo_edge_index, trans_edge_index, trans_w,
           W1_0, b1_0, W2_0, W1_1, b1_1, W2_1):
  npad = EP - EE
  pad_src = jnp.arange(npad, dtype=jnp.int32) % NN
  pad_dst = NN + jnp.arange(npad, dtype=jnp.int32) % (NP - NN)

  def prep(ei):
    s = jnp.concatenate([ei[0], pad_src]).reshape(EP // KC, KC)
    d = jnp.concatenate([ei[1], pad_dst]).reshape(EP // KC, KC)
    return s, d

  src_g2, dst_g2 = prep(geo_edge_index)
  src_t2, dst_t2 = prep(trans_edge_index)
  w_t2 = jnp.concatenate(
      [trans_w, jnp.zeros((npad,), jnp.float32)]).reshape(EP // KC, KC)
  x4 = jnp.stack([loc_feat[:, q * QD:(q + 1) * QD] for q in range(NQ)])
  x2 = jnp.stack([loc_feat[:, :DH], loc_feat[:, DH:]])
  b1_0r = b1_0.reshape(1, HH)
  b1_1r = b1_1.reshape(1, HH)
  w2_0r = W2_0.reshape(1, HH)
  w2_1r = W2_1.reshape(1, HH)

  w1_0b = W1_0.astype(jnp.bfloat16)
  w1_1b = W1_1.astype(jnp.bfloat16)

  fl1 = jnp.ones((16,), jnp.int32)
  fl0 = jnp.zeros((16,), jnp.int32)
  gsum2, tsum2, deg2 = _agg(x4, src_g2, dst_g2, src_t2, dst_t2, w_t2, fl1)
  geo1, hsum1 = _dense(x2, gsum2, tsum2, deg2, w1_0b, b1_0r)
  x2, x4 = _combine(hsum1, w2_0r, geo1, tsum2, True)

  gsum2, tsum2, _ = _agg(x4, src_g2, dst_g2, src_t2, dst_t2, w_t2, fl0)
  geo2, hsum2 = _dense(x2, gsum2, tsum2, deg2, w1_1b, b1_1r)
  (x2,) = _combine(hsum2, w2_1r, geo2, tsum2, False)

  return jnp.moveaxis(x2, 0, 1).reshape(NN, DD)


# speedup vs baseline: 1.4304x; 1.0058x over previous
import functools

import jax
import jax.numpy as jnp
from jax import lax
from jax.experimental import pallas as pl
from jax.experimental.pallas import tpu as pltpu
from jax.experimental.pallas import tpu_sc as plsc

NN = 10000
DD = 256
DH = 128
QD = 64
NQ = DD // QD
HH = 1024
EE = 160000
NC = 2
NS = 16
NP = 10240
RPT = NP // NS
KC = 128
EP = 163840
CPT = EP // (NS * KC)
NBUF = 5
PD = 4


@functools.cache
def _mesh():
  return plsc.VectorSubcoreMesh(
      core_axis_name="c", subcore_axis_name="s", num_cores=NC, num_subcores=NS)


def _agg_body(x4, src_g2, dst_g2, src_t2, dst_t2, w_t2, dflag,
              gsum4, tsum4, deg_out,
              acc_sh, idx_all, dst_all, w_all, flag_v, *bufs):
  rows = bufs[:NBUF]
  semg = bufs[NBUF:2 * NBUF]
  sems = bufs[2 * NBUF:3 * NBUF]
  cid = lax.axis_index("c")
  sid = lax.axis_index("s")
  rbase = sid * RPT
  cbase = sid * CPT

  def fill(buf, val):
    v = jnp.full((16,), val, jnp.float32)

    def row(e, _):
      for j in range(QD // 16):
        buf[e, pl.ds(j * 16, 16)] = v
      return 0

    lax.fori_loop(0, KC, row, 0)

  def zero_acc():
    fill(rows[0], 0.0)
    for i in range(RPT // KC):
      pltpu.sync_copy(rows[0], acc_sh.at[pl.ds(rbase + i * KC, KC)])

  def flush(out, q):
    pltpu.sync_copy(
        acc_sh.at[pl.ds(rbase, RPT)],
        out.at[cid].at[pl.ds(rbase, RPT), pl.ds(q * QD, QD)])

  def scale_buf(buf, c):
    def group(g, _):
      w16 = w_all[c, pl.ds(g * 16, 16)]
      for lane in range(16):
        w = w16[lane]
        for j in range(QD // 16):
          buf[g * 16 + lane, pl.ds(j * 16, 16)] = (
              buf[g * 16 + lane, pl.ds(j * 16, 16)] * w)
      return 0

    lax.fori_loop(0, KC // 16, group, 0)

  def wait_gather(xq, b):
    pltpu.make_async_copy(xq.at[idx_all.at[0]], rows[b], semg[b]).wait()

  def wait_scatter(b):
    pltpu.make_async_copy(rows[b], acc_sh.at[dst_all.at[0]], sems[b]).wait()

  def quarter_pass(xq, scale):
    for b in range(PD):
      pltpu.async_copy(xq.at[idx_all.at[b]], rows[b], semg[b])

    def quad(q, _):
      for b in range(NBUF):
        c = q * NBUF + b
        wait_gather(xq, b)
        if scale:
          scale_buf(rows[b], c)
        pltpu.async_copy(rows[b], acc_sh.at[dst_all.at[c]], sems[b],
                         add=True)
        cp = c + PD
        bp = (b + PD) % NBUF

        @pl.when(cp < CPT)
        def _():
          @pl.when(cp >= NBUF)
          def _():
            wait_scatter(bp)
          pltpu.async_copy(xq.at[idx_all.at[cp]], rows[bp], semg[bp])
      return 0

    lax.fori_loop(0, CPT // NBUF, quad, 0)
    for b in range(NBUF):
      wait_scatter(b)

  def relation(src2, dst2, out, scale):
    pltpu.sync_copy(src2.at[pl.ds(cbase, CPT)], idx_all)
    pltpu.sync_copy(dst2.at[pl.ds(cbase, CPT)], dst_all)
    if scale:
      pltpu.sync_copy(w_t2.at[pl.ds(cbase, CPT)], w_all)
    for q in range(NC):
      qidx = cid * NC + q
      zero_acc()
      plsc.subcore_barrier()
      quarter_pass(x4.at[qidx], scale)
      plsc.subcore_barrier()
      flush(out, q)
      plsc.subcore_barrier()

  relation(src_g2, dst_g2, gsum4, False)

  pltpu.sync_copy(dflag, flag_v)
  do_deg = jnp.sum(flag_v[...]) > 0

  @pl.when(do_deg)
  def _():
    zero_acc()
    fill(rows[1], 1.0)

  plsc.subcore_barrier()

  @pl.when(do_deg)
  def _():
    def dchunk(i, _):
      c = cid * (CPT // 2) + i

      @pl.when(i >= 2)
      def _():
        wait_scatter(1)

      pltpu.async_copy(rows[1], acc_sh.at[dst_all.at[c]], sems[1], add=True)
      return 0

    lax.fori_loop(0, CPT // 2, dchunk, 0)
    wait_scatter(1)
    wait_scatter(1)

  plsc.subcore_barrier()

  @pl.when(do_deg)
  def _():
    flush(deg_out, 0)

  plsc.subcore_barrier()

  relation(src_t2, dst_t2, tsum4, True)


def _make_agg():
  out_type = [
      jax.ShapeDtypeStruct((NC, NP, DH), jnp.float32),
      jax.ShapeDtypeStruct((NC, NP, DH), jnp.float32),
      jax.ShapeDtypeStruct((NC, NP, DH), jnp.float32),
  ]
  scratch = (
      [pltpu.VMEM_SHARED((NP, QD), jnp.float32)]
      + [pltpu.VMEM((CPT, KC), jnp.int32)] * 2
      + [pltpu.VMEM((CPT, KC), jnp.float32)]
      + [pltpu.VMEM((16,), jnp.int32)]
      + [pltpu.VMEM((KC, QD), jnp.float32)] * NBUF
      + [pltpu.SemaphoreType.DMA] * (2 * NBUF)
  )
  return pl.kernel(_agg_body, out_type=out_type, mesh=_mesh(),
                   scratch_types=scratch, name="sc_agg",
                   compiler_params=pltpu.CompilerParams(
                       use_tc_tiling_on_sc=False,
                       needs_layout_passes=False))


_agg = lambda *a: _make_agg_cached()(*a)
_make_agg_cached = functools.cache(_make_agg)

TT = 400
GRID = NN // TT


def _dense_body(x_lo, x_hi, g_lo, g_hi, t_lo, t_hi, deg_a, deg_b, w1, b1,
                geo_out, hsum_out):
  i = pl.program_id(0)
  x = jnp.concatenate([x_lo[0], x_hi[0]], axis=1)
  gs = jnp.concatenate([g_lo[0], g_hi[0]], axis=1)
  ts = jnp.concatenate([t_lo[0], t_hi[0]], axis=1)
  invd = 1.0 / (deg_a[0, :, 0:1] + deg_b[0, :, 0:1] + 1.0)
  geo = (gs + x) * invd
  geo_out[...] = geo
  w1b = w1[...]
  hg = jnp.tanh(jnp.dot(geo.astype(jnp.bfloat16), w1b,
                        preferred_element_type=jnp.float32) + b1[...])
  ht = jnp.tanh(jnp.dot(ts.astype(jnp.bfloat16), w1b,
                        preferred_element_type=jnp.float32) + b1[...])
  s = jnp.concatenate([jnp.sum(hg, 0, keepdims=True),
                       jnp.sum(ht, 0, keepdims=True)], axis=0)

  @pl.when(i == 0)
  def _():
    hsum_out[...] = s

  @pl.when(i > 0)
  def _():
    hsum_out[...] += s


def _half(c):
  return pl.BlockSpec((1, TT, DH), lambda i, c=c: (c, i, 0))


def _dense(x2, gsum2, tsum2, deg2, w1, b1r):
  return pl.pallas_call(
      _dense_body,
      grid=(GRID,),
      in_specs=[_half(0), _half(1), _half(0), _half(1), _half(0), _half(1),
                _half(0), _half(1),
                pl.BlockSpec((DD, HH), lambda i: (0, 0)),
                pl.BlockSpec((1, HH), lambda i: (0, 0))],
      out_specs=[pl.BlockSpec((TT, DD), lambda i: (i, 0)),
                 pl.BlockSpec((2, HH), lambda i: (0, 0))],
      out_shape=[jax.ShapeDtypeStruct((NN, DD), jnp.float32),
                 jax.ShapeDtypeStruct((2, HH), jnp.float32)],
  )(x2, x2, gsum2, gsum2, tsum2, tsum2, deg2, deg2, w1, b1r)


def _combine_body(emit_q, hsum, w2r, geo, t_lo, t_hi, *outs):
  wm = jnp.sum(hsum[...] * w2r[...], axis=1) / NN
  m = jnp.max(wm)
  e = jnp.exp(wm - m)
  beta = e / jnp.sum(e)
  g = geo[...]
  lo = beta[0] * g[:, :DH] + beta[1] * t_lo[0]
  hi = beta[0] * g[:, DH:] + beta[1] * t_hi[0]
  outs[0][0] = lo
  outs[0][1] = hi
  if emit_q:
    outs[1][0] = lo[:, :QD]
    outs[1][1] = lo[:, QD:]
    outs[1][2] = hi[:, :QD]
    outs[1][3] = hi[:, QD:]


def _combine(hsum, w2r, geo, tsum2, emit_q):
  out_specs = [pl.BlockSpec((NC, TT, DH), lambda i: (0, i, 0))]
  out_shape = [jax.ShapeDtypeStruct((NC, NN, DH), jnp.float32)]
  if emit_q:
    out_specs.append(pl.BlockSpec((NQ, TT, QD), lambda i: (0, i, 0)))
    out_shape.append(jax.ShapeDtypeStruct((NQ, NN, QD), jnp.float32))
  return pl.pallas_call(
      functools.partial(_combine_body, emit_q),
      grid=(GRID,),
      in_specs=[pl.BlockSpec((2, HH), lambda i: (0, 0)),
                pl.BlockSpec((1, HH), lambda i: (0, 0)),
                pl.BlockSpec((TT, DD), lambda i: (i, 0)),
                _half(0), _half(1)],
      out_specs=out_specs,
      out_shape=out_shape,
  )(hsum, w2r, geo, tsum2, tsum2)


def kernel(loc_feat, geo_edge_index, trans_edge_index, trans_w,
           W1_0, b1_0, W2_0, W1_1, b1_1, W2_1):
  npad = EP - EE
  pad_src = jnp.arange(npad, dtype=jnp.int32) % NN
  pad_dst = NN + jnp.arange(npad, dtype=jnp.int32) % (NP - NN)

  def prep(ei):
    s = jnp.concatenate([ei[0], pad_src]).reshape(EP // KC, KC)
    d = jnp.concatenate([ei[1], pad_dst]).reshape(EP // KC, KC)
    return s, d

  src_g2, dst_g2 = prep(geo_edge_index)
  src_t2, dst_t2 = prep(trans_edge_index)
  w_t2 = jnp.concatenate(
      [trans_w, jnp.zeros((npad,), jnp.float32)]).reshape(EP // KC, KC)
  x4 = jnp.stack([loc_feat[:, q * QD:(q + 1) * QD] for q in range(NQ)])
  x2 = jnp.stack([loc_feat[:, :DH], loc_feat[:, DH:]])
  b1_0r = b1_0.reshape(1, HH)
  b1_1r = b1_1.reshape(1, HH)
  w2_0r = W2_0.reshape(1, HH)
  w2_1r = W2_1.reshape(1, HH)

  w1_0b = W1_0.astype(jnp.bfloat16)
  w1_1b = W1_1.astype(jnp.bfloat16)

  fl1 = jnp.ones((16,), jnp.int32)
  fl0 = jnp.zeros((16,), jnp.int32)
  gsum2, tsum2, deg2 = _agg(x4, src_g2, dst_g2, src_t2, dst_t2, w_t2, fl1)
  geo1, hsum1 = _dense(x2, gsum2, tsum2, deg2, w1_0b, b1_0r)
  x2, x4 = _combine(hsum1, w2_0r, geo1, tsum2, True)

  gsum2, tsum2, _ = _agg(x4, src_g2, dst_g2, src_t2, dst_t2, w_t2, fl0)
  geo2, hsum2 = _dense(x2, gsum2, tsum2, deg2, w1_1b, b1_1r)
  (x2,) = _combine(hsum2, w2_1r, geo2, tsum2, False)

  return jnp.moveaxis(x2, 0, 1).reshape(NN, DD)
